# baseline (device time: 21014 ns/iter reference)
import jax
import jax.numpy as jnp
from jax import lax
from jax.experimental import pallas as pl
from jax.experimental.pallas import tpu as pltpu

N_DEV = 4


def kernel(x):
    m, n = x.shape
    ch = m // N_DEV

    def body(x_ref, out_ref, xb, rs_buf, ag_src,
             rs_send_sems, rs_recv_sems, ag_send_sems, ag_recv_sems):
        my = lax.axis_index("i")

        barrier_sem = pltpu.get_barrier_semaphore()
        for k in range(1, N_DEV):
            pl.semaphore_signal(
                barrier_sem, inc=1,
                device_id=((my + k) % N_DEV,),
                device_id_type=pl.DeviceIdType.MESH,
            )
        pl.semaphore_wait(barrier_sem, N_DEV - 1)

        xb[:, :] = x_ref[:, :].astype(jnp.bfloat16)

        rs_rdmas = []
        for k in range(1, N_DEV):
            dst = (my + k) % N_DEV
            rdma = pltpu.make_async_remote_copy(
                src_ref=xb.at[pl.ds(dst * ch, ch), :],
                dst_ref=rs_buf.at[k - 1],
                send_sem=rs_send_sems.at[k - 1],
                recv_sem=rs_recv_sems.at[k - 1],
                device_id=(dst,),
                device_id_type=pl.DeviceIdType.MESH,
            )
            rdma.start()
            rs_rdmas.append(rdma)

        for rdma in rs_rdmas:
            rdma.wait_recv()

        ag_src[:, :] = (
            xb[pl.ds(my * ch, ch), :]
            + rs_buf[0, :, :] + rs_buf[1, :, :] + rs_buf[2, :, :]
        )

        ag_rdmas = []
        for k in range(1, N_DEV):
            dst = (my + k) % N_DEV
            rdma = pltpu.make_async_remote_copy(
                src_ref=ag_src,
                dst_ref=out_ref.at[pl.ds(my * ch, ch), :],
                send_sem=ag_send_sems.at[k - 1],
                recv_sem=ag_recv_sems.at[k - 1],
                device_id=(dst,),
                device_id_type=pl.DeviceIdType.MESH,
            )
            rdma.start()
            ag_rdmas.append(rdma)

        out_ref[pl.ds(my * ch, ch), :] = ag_src[:, :]

        for rdma in ag_rdmas:
            rdma.wait_recv()

        for rdma in rs_rdmas:
            rdma.wait_send()
        for rdma in ag_rdmas:
            rdma.wait_send()

    return pl.pallas_call(
        body,
        out_shape=jax.ShapeDtypeStruct((m, n), jnp.bfloat16),
        in_specs=[pl.BlockSpec(memory_space=pltpu.VMEM)],
        out_specs=pl.BlockSpec(memory_space=pltpu.VMEM),
        scratch_shapes=[
            pltpu.VMEM((m, n), jnp.bfloat16),
            pltpu.VMEM((N_DEV - 1, ch, n), jnp.bfloat16),
            pltpu.VMEM((ch, n), jnp.bfloat16),
            pltpu.SemaphoreType.DMA((N_DEV - 1,)),
            pltpu.SemaphoreType.DMA((N_DEV - 1,)),
            pltpu.SemaphoreType.DMA((N_DEV - 1,)),
            pltpu.SemaphoreType.DMA((N_DEV - 1,)),
        ],
        compiler_params=pltpu.CompilerParams(collective_id=0),
    )(x)


# device time: 18984 ns/iter; 1.1069x vs baseline; 1.1069x over previous
import jax
import jax.numpy as jnp
from jax import lax
from jax.experimental import pallas as pl
from jax.experimental.pallas import tpu as pltpu

N_DEV = 4
S = 2


def kernel(x):
    m, n = x.shape
    ch = m // N_DEV
    sub = ch // S

    def body(x_ref, out_ref, xb, rs_buf, ag_src,
             rs_send_sems, rs_recv_sems, ag_send_sems, ag_recv_sems):
        my = lax.axis_index("i")

        barrier_sem = pltpu.get_barrier_semaphore()
        for k in range(1, N_DEV):
            pl.semaphore_signal(
                barrier_sem, inc=1,
                device_id=((my + k) % N_DEV,),
                device_id_type=pl.DeviceIdType.MESH,
            )
        pl.semaphore_wait(barrier_sem, N_DEV - 1)

        xb[:, :] = x_ref[:, :].astype(jnp.bfloat16)

        rs_rdmas = {}
        for s in range(S):
            for k in range(1, N_DEV):
                dst = (my + k) % N_DEV
                rdma = pltpu.make_async_remote_copy(
                    src_ref=xb.at[pl.ds(dst * ch + s * sub, sub), :],
                    dst_ref=rs_buf.at[k - 1, pl.ds(s * sub, sub), :],
                    send_sem=rs_send_sems.at[k - 1, s],
                    recv_sem=rs_recv_sems.at[k - 1, s],
                    device_id=(dst,),
                    device_id_type=pl.DeviceIdType.MESH,
                )
                rdma.start()
                rs_rdmas[(k, s)] = rdma

        ag_rdmas = []
        for s in range(S):
            for k in range(1, N_DEV):
                rs_rdmas[(k, s)].wait_recv()
            rows = pl.ds(s * sub, sub)
            ag_src[rows, :] = (
                xb[pl.ds(my * ch + s * sub, sub), :]
                + rs_buf[0, rows, :]
                + rs_buf[1, rows, :]
                + rs_buf[2, rows, :]
            )
            for k in range(1, N_DEV):
                dst = (my + k) % N_DEV
                rdma = pltpu.make_async_remote_copy(
                    src_ref=ag_src.at[rows, :],
                    dst_ref=out_ref.at[pl.ds(my * ch + s * sub, sub), :],
                    send_sem=ag_send_sems.at[k - 1, s],
                    recv_sem=ag_recv_sems.at[k - 1, s],
                    device_id=(dst,),
                    device_id_type=pl.DeviceIdType.MESH,
                )
                rdma.start()
                ag_rdmas.append(rdma)

        out_ref[pl.ds(my * ch, ch), :] = ag_src[:, :]

        for rdma in ag_rdmas:
            rdma.wait_recv()

        for rdma in rs_rdmas.values():
            rdma.wait_send()
        for rdma in ag_rdmas:
            rdma.wait_send()

    return pl.pallas_call(
        body,
        out_shape=jax.ShapeDtypeStruct((m, n), jnp.bfloat16),
        in_specs=[pl.BlockSpec(memory_space=pltpu.VMEM)],
        out_specs=pl.BlockSpec(memory_space=pltpu.VMEM),
        scratch_shapes=[
            pltpu.VMEM((m, n), jnp.bfloat16),
            pltpu.VMEM((N_DEV - 1, ch, n), jnp.bfloat16),
            pltpu.VMEM((ch, n), jnp.bfloat16),
            pltpu.SemaphoreType.DMA((N_DEV - 1, S)),
            pltpu.SemaphoreType.DMA((N_DEV - 1, S)),
            pltpu.SemaphoreType.DMA((N_DEV - 1, S)),
            pltpu.SemaphoreType.DMA((N_DEV - 1, S)),
        ],
        compiler_params=pltpu.CompilerParams(collective_id=0),
    )(x)
